# baseline (speedup 1.0000x reference)
import functools

import jax
import jax.numpy as jnp
import numpy as np
from jax import lax
from jax.experimental import pallas as pl
from jax.experimental.pallas import tpu as pltpu
from jax.experimental.pallas import tpu_sc as plsc

_B = 1024
_BANK = 4096
_S, _M = 64, 96
_NC = 2
_NS = 16
_NW = _NC * _NS
_SPW = _S // _NW
_CH = 8
_MCHUNK = _M // _CH
_NCHUNK = _SPW * _MCHUNK
_NPAIR = _NCHUNK // 2


def _gather_body(table_hbm, idx_hbm, out_hbm, idx_v,
                 in0, in1, out0, out1, gs0, gs1, ps0, ps1):
    wid = lax.axis_index("s") * _NC + lax.axis_index("c")
    s_base = wid * _SPW

    def src(c):
        return s_base + c // _MCHUNK, (c % _MCHUNK) * _CH

    def g_start(c, buf, sem):
        s, m0 = src(c)
        pltpu.async_copy(table_hbm.at[s, pl.ds(m0, _CH)], buf, sem)

    def g_wait(c, buf, sem):
        s, m0 = src(c)
        pltpu.make_async_copy(table_hbm.at[s, pl.ds(m0, _CH)], buf, sem).wait()

    def p_start(c, buf, sem):
        s, m0 = src(c)
        pltpu.async_copy(buf, out_hbm.at[s, pl.ds(m0, _CH)], sem)

    def p_wait(c, buf, sem):
        s, m0 = src(c)
        pltpu.make_async_copy(buf, out_hbm.at[s, pl.ds(m0, _CH)], sem).wait()

    def compact(inb, outb):
        for k2 in range(_B // 32):
            ivs = [idx_v[pl.ds(16 * (2 * k2 + j), 16)] for j in range(2)]
            gs = []
            for r in range(_CH):
                rv = jnp.full((16,), r, jnp.int32)
                for j in range(2):
                    gs.append(plsc.load_gather(inb, [rv, ivs[j]]))
            i = 0
            for r in range(_CH):
                for j in range(2):
                    outb[r, pl.ds(16 * (2 * k2 + j), 16)] = gs[i]
                    i += 1

    g_start(0, in0, gs0)
    g_start(1, in1, gs1)
    pltpu.sync_copy(idx_hbm, idx_v)

    def pair(t, carry):
        c0 = 2 * t
        c1 = c0 + 1

        g_wait(c0, in0, gs0)
        pl.when(t > 0)(lambda: p_wait(c0 - 2, out0, ps0))
        compact(in0, out0)
        p_start(c0, out0, ps0)
        pl.when(t + 1 < _NPAIR)(lambda: g_start(c0 + 2, in0, gs0))

        g_wait(c1, in1, gs1)
        pl.when(t > 0)(lambda: p_wait(c1 - 2, out1, ps1))
        compact(in1, out1)
        p_start(c1, out1, ps1)
        pl.when(t + 1 < _NPAIR)(lambda: g_start(c1 + 2, in1, gs1))
        return carry

    lax.fori_loop(0, _NPAIR, pair, 0)
    p_wait(_NCHUNK - 2, out0, ps0)
    p_wait(_NCHUNK - 1, out1, ps1)


@functools.partial(
    pl.kernel,
    mesh=plsc.VectorSubcoreMesh(core_axis_name="c", subcore_axis_name="s"),
    out_type=jax.ShapeDtypeStruct((_S, _M, _B), jnp.float32),
    scratch_types=[
        pltpu.VMEM((_B,), jnp.int32),
        pltpu.VMEM((_CH, _BANK), jnp.float32),
        pltpu.VMEM((_CH, _BANK), jnp.float32),
        pltpu.VMEM((_CH, _B), jnp.float32),
        pltpu.VMEM((_CH, _B), jnp.float32),
        pltpu.SemaphoreType.DMA,
        pltpu.SemaphoreType.DMA,
        pltpu.SemaphoreType.DMA,
        pltpu.SemaphoreType.DMA,
    ],
    compiler_params=pltpu.CompilerParams(needs_layout_passes=False),
)
def _gather(table_hbm, idx_hbm, out_hbm, idx_v,
            in0, in1, out0, out1, gs0, gs1, ps0, ps1):
    _gather_body(table_hbm, idx_hbm, out_hbm, idx_v,
                 in0, in1, out0, out1, gs0, gs1, ps0, ps1)


_IDX = np.asarray(
    jax.random.randint(jax.random.key(42), (_B,), 0, _BANK)
).astype(np.int32)


def kernel(x, y, audio, pose, dummy):
    idx = jnp.asarray(_IDX)
    table_t = pose.transpose(1, 2, 0)
    out_t = _gather(table_t, idx)
    out = out_t.transpose(2, 0, 1)
    return out, jnp.sum(dummy)


# speedup vs baseline: 1.1276x; 1.1276x over previous
import functools

import jax
import jax.numpy as jnp
from jax import lax
from jax.experimental import pallas as pl
from jax.experimental.pallas import tpu as pltpu
from jax.experimental.pallas import tpu_sc as plsc

_B = 1024
_BANK = 4096
_S, _M = 64, 96
_NC = 2
_NS = 16
_NW = _NC * _NS
_SPW = _S // _NW
_CH = 8
_MCHUNK = _M // _CH
_NCHUNK = _SPW * _MCHUNK
_NPAIR = _NCHUNK // 2


def _gather_body(table_hbm, idx_hbm, out_hbm, idx_v,
                 in0, in1, out0, out1, gs0, gs1, ps0, ps1):
    wid = lax.axis_index("s") * _NC + lax.axis_index("c")
    s_base = wid * _SPW

    def src(c):
        return s_base + c // _MCHUNK, (c % _MCHUNK) * _CH

    def g_start(c, buf, sem):
        s, m0 = src(c)
        pltpu.async_copy(table_hbm.at[s, pl.ds(m0, _CH)], buf, sem)

    def g_wait(c, buf, sem):
        s, m0 = src(c)
        pltpu.make_async_copy(table_hbm.at[s, pl.ds(m0, _CH)], buf, sem).wait()

    def p_start(c, buf, sem):
        s, m0 = src(c)
        pltpu.async_copy(buf, out_hbm.at[s, pl.ds(m0, _CH)], sem)

    def p_wait(c, buf, sem):
        s, m0 = src(c)
        pltpu.make_async_copy(buf, out_hbm.at[s, pl.ds(m0, _CH)], sem).wait()

    def compact(inb, outb):
        def kbody(k2, carry):
            base = 32 * k2
            ivs = [idx_v[pl.ds(base + 16 * j, 16)] for j in range(2)]
            gs = []
            for r in range(_CH):
                rv = jnp.full((16,), r, jnp.int32)
                for j in range(2):
                    gs.append(plsc.load_gather(inb, [rv, ivs[j]]))
            i = 0
            for r in range(_CH):
                for j in range(2):
                    outb[r, pl.ds(base + 16 * j, 16)] = gs[i]
                    i += 1
            return carry

        lax.fori_loop(0, _B // 32, kbody, 0)

    g_start(0, in0, gs0)
    g_start(1, in1, gs1)
    pltpu.sync_copy(idx_hbm, idx_v)

    def pair(t, carry):
        c0 = 2 * t
        c1 = c0 + 1

        g_wait(c0, in0, gs0)
        pl.when(t > 0)(lambda: p_wait(c0 - 2, out0, ps0))
        compact(in0, out0)
        p_start(c0, out0, ps0)
        pl.when(t + 1 < _NPAIR)(lambda: g_start(c0 + 2, in0, gs0))

        g_wait(c1, in1, gs1)
        pl.when(t > 0)(lambda: p_wait(c1 - 2, out1, ps1))
        compact(in1, out1)
        p_start(c1, out1, ps1)
        pl.when(t + 1 < _NPAIR)(lambda: g_start(c1 + 2, in1, gs1))
        return carry

    lax.fori_loop(0, _NPAIR, pair, 0)
    p_wait(_NCHUNK - 2, out0, ps0)
    p_wait(_NCHUNK - 1, out1, ps1)


@functools.partial(
    pl.kernel,
    mesh=plsc.VectorSubcoreMesh(core_axis_name="c", subcore_axis_name="s"),
    out_type=jax.ShapeDtypeStruct((_S, _M, _B), jnp.float32),
    scratch_types=[
        pltpu.VMEM((_B,), jnp.int32),
        pltpu.VMEM((_CH, _BANK), jnp.float32),
        pltpu.VMEM((_CH, _BANK), jnp.float32),
        pltpu.VMEM((_CH, _B), jnp.float32),
        pltpu.VMEM((_CH, _B), jnp.float32),
        pltpu.SemaphoreType.DMA,
        pltpu.SemaphoreType.DMA,
        pltpu.SemaphoreType.DMA,
        pltpu.SemaphoreType.DMA,
    ],
    compiler_params=pltpu.CompilerParams(needs_layout_passes=False),
)
def _gather(table_hbm, idx_hbm, out_hbm, idx_v,
            in0, in1, out0, out1, gs0, gs1, ps0, ps1):
    _gather_body(table_hbm, idx_hbm, out_hbm, idx_v,
                 in0, in1, out0, out1, gs0, gs1, ps0, ps1)


def kernel(x, y, audio, pose, dummy):
    idx = jax.random.randint(
        jax.random.key(42), (y.shape[0],), 0, pose.shape[0]
    ).astype(jnp.int32)
    table_t = pose.transpose(1, 2, 0)
    out_t = _gather(table_t, idx)
    out = out_t.transpose(2, 0, 1)
    return out, jnp.sum(dummy)
